# baseline (speedup 1.0000x reference)
import functools

import jax
import jax.numpy as jnp
from jax import lax
from jax.experimental import pallas as pl
from jax.experimental.pallas import tpu as pltpu
from jax.experimental.pallas import tpu_sc as plsc

NGRAPH = 20
NODES = 500
NPAD = 512
KNN = 16
FEAT = 128
BIG = 1e30
_HI = lax.Precision.HIGHEST

_SC_CORES = 2
_SC_SUBCORES = 16
_SC_WORKERS = _SC_CORES * _SC_SUBCORES
_GATHER_CHUNK = 128
GROW = 128


def _dot(a, b, prec=lax.Precision.DEFAULT):
    return lax.dot_general(a, b, (((1,), (0,)), ((), ())),
                           preferred_element_type=jnp.float32, precision=prec)


def _dot_t(a, b, prec=_HI):
    return lax.dot_general(a, b, (((1,), (1,)), ((), ())),
                           preferred_element_type=jnp.float32, precision=prec)


_TKROWS = 64


def _topk_neighbors(d2_ref, nbr_ref, base):
    col = lax.broadcasted_iota(jnp.int32, (_TKROWS, NPAD), 1)
    lane_k = lax.broadcasted_iota(jnp.int32, (_TKROWS, KNN), 1)

    def chunk(ci, carry):
        d2 = d2_ref[pl.ds(ci * _TKROWS, _TKROWS), :]
        idx_acc = jnp.zeros((_TKROWS, KNN), jnp.int32)
        for t in range(KNN):
            m = jnp.min(d2, axis=1, keepdims=True)
            idx = jnp.min(jnp.where(d2 == m, col, jnp.int32(2 ** 30)),
                          axis=1, keepdims=True)
            idx_acc = jnp.where(lane_k == t, idx + base, idx_acc)
            d2 = jnp.where(col == idx, BIG, d2)
        nbr_ref[0, pl.ds(ci * _TKROWS, _TKROWS), :] = idx_acc
        return carry

    lax.fori_loop(0, NPAD // _TKROWS, chunk, 0, unroll=False)


def _pair_dist(feat, d2_ref):
    g = _dot_t(feat, feat)
    sq = jnp.sum(feat * feat, axis=1, keepdims=True)
    ones = jnp.ones((NPAD, 1), jnp.float32)
    sqr = _dot_t(ones, sq)
    d2 = sqr - 2.0 * g
    col = lax.broadcasted_iota(jnp.int32, (NPAD, NPAD), 1)
    row = lax.broadcasted_iota(jnp.int32, (NPAD, NPAD), 0)
    d2_ref[...] = jnp.where(col >= NODES, BIG,
                            jnp.where(col == row, d2 + 1e9, d2))


def _stage1_kernel(x_ref, w1b_ref, w1c_ref, bias1_ref,
                   nbr_ref, b0_ref, c0_ref, d2_ref):
    g = pl.program_id(0)
    x = x_ref[...]
    _pair_dist(x[:, 0:2], d2_ref)
    _topk_neighbors(d2_ref, nbr_ref, g * NPAD)
    b0_ref[...] = _dot(x, w1b_ref[...])
    c0_ref[...] = _dot(x, w1c_ref[...]) + bias1_ref[0:1, :]


def _edge_mlp_tail(h1, w2_ref, bias2_ref, w3_ref, bias3_ref):
    h = jnp.maximum(h1, 0.0)
    h = jnp.maximum(_dot(h, w2_ref[...]) + bias2_ref[0:1, :], 0.0)
    h = jnp.maximum(_dot(h, w3_ref[...]) + bias3_ref[0:1, :], 0.0)
    acc = h[0:NPAD]
    for k in range(1, KNN):
        acc = acc + h[k * NPAD:(k + 1) * NPAD]
    return acc * (1.0 / KNN)


def _stage2_kernel(g0_ref, c0_ref, x_ref,
                   w2_ref, bias2_ref, w3_ref, bias3_ref,
                   w1b_ref, w1c_ref, bias1_ref,
                   nbr_ref, b1_ref, c1_ref, xb1_ref, d2_ref):
    g = pl.program_id(0)
    c0 = c0_ref[...]
    l0 = c0.shape[1]
    h1 = g0_ref[:, 0:l0] + jnp.concatenate([c0] * KNN, axis=0)
    conv = _edge_mlp_tail(h1, w2_ref, bias2_ref, w3_ref, bias3_ref)
    xb1 = jnp.concatenate([conv, x_ref[...]], axis=1)
    xb1_ref[...] = xb1
    _pair_dist(xb1, d2_ref)
    _topk_neighbors(d2_ref, nbr_ref, g * NPAD)
    b1_ref[...] = _dot(xb1, w1b_ref[...])
    c1_ref[...] = _dot(xb1, w1c_ref[...]) + bias1_ref[0:1, :]


def _stage3_kernel(g1_ref, c1_ref, xb1_ref,
                   w2_ref, bias2_ref, w3_ref, bias3_ref,
                   wc_ref, bc_ref, out_ref):
    c1 = c1_ref[...]
    l1 = c1.shape[1]
    h1 = g1_ref[:, 0:l1] + jnp.concatenate([c1] * KNN, axis=0)
    conv = _edge_mlp_tail(h1, w2_ref, bias2_ref, w3_ref, bias3_ref)
    xb2 = jnp.concatenate([conv, xb1_ref[...]], axis=1)
    rows = lax.broadcasted_iota(jnp.int32, (NPAD, 1), 0)
    xb2 = jnp.where(rows < NODES, xb2, 0.0)
    pooled = jnp.sum(xb2, axis=0, keepdims=True) * (1.0 / NODES)
    out_ref[0] = _dot(pooled, wc_ref[...]) + bc_ref[0:1, :]


def _graph_spec(cols):
    return pl.BlockSpec((NPAD, cols), lambda g: (g, 0))


def _edge_spec(cols):
    return pl.BlockSpec((NPAD * KNN, cols), lambda g: (g, 0))


def _full_spec(rows, cols):
    return pl.BlockSpec((rows, cols), lambda g: (0, 0))


_NBR_SPEC = pl.BlockSpec((1, NPAD, KNN), lambda g: (g, 0, 0))
_CPARAMS = pltpu.CompilerParams(dimension_semantics=("arbitrary",))


def _stage1(xp, w1b, w1c, bias1):
    l0 = w1c.shape[1]
    return pl.pallas_call(
        _stage1_kernel,
        grid=(NGRAPH,),
        in_specs=[_graph_spec(FEAT), _full_spec(FEAT, GROW),
                  _full_spec(FEAT, l0), _full_spec(8, l0)],
        out_specs=[_NBR_SPEC, _graph_spec(GROW), _graph_spec(l0)],
        out_shape=[
            jax.ShapeDtypeStruct((NGRAPH, NPAD, KNN), jnp.int32),
            jax.ShapeDtypeStruct((NGRAPH * NPAD, GROW), jnp.float32),
            jax.ShapeDtypeStruct((NGRAPH * NPAD, l0), jnp.float32),
        ],
        scratch_shapes=[pltpu.VMEM((NPAD, NPAD), jnp.float32)],
        compiler_params=_CPARAMS,
    )(xp, w1b, w1c, bias1)


def _stage2(g0, c0, xp, w2, bias2, w3, bias3, w1b, w1c, bias1):
    l0 = c0.shape[1]
    l1 = w1c.shape[1]
    d1 = FEAT + l0
    return pl.pallas_call(
        _stage2_kernel,
        grid=(NGRAPH,),
        in_specs=[_edge_spec(GROW), _graph_spec(l0), _graph_spec(FEAT),
                  _full_spec(l0, l0), _full_spec(8, l0),
                  _full_spec(l0, l0), _full_spec(8, l0),
                  _full_spec(d1, GROW), _full_spec(d1, l1), _full_spec(8, l1)],
        out_specs=[_NBR_SPEC, _graph_spec(GROW), _graph_spec(l1),
                   _graph_spec(d1)],
        out_shape=[
            jax.ShapeDtypeStruct((NGRAPH, NPAD, KNN), jnp.int32),
            jax.ShapeDtypeStruct((NGRAPH * NPAD, GROW), jnp.float32),
            jax.ShapeDtypeStruct((NGRAPH * NPAD, l1), jnp.float32),
            jax.ShapeDtypeStruct((NGRAPH * NPAD, d1), jnp.float32),
        ],
        scratch_shapes=[pltpu.VMEM((NPAD, NPAD), jnp.float32)],
        compiler_params=_CPARAMS,
    )(g0, c0, xp, w2, bias2, w3, bias3, w1b, w1c, bias1)


def _stage3(g1, c1, xb1, w2, bias2, w3, bias3, wc, bc):
    l1 = c1.shape[1]
    d1 = xb1.shape[1]
    d2 = d1 + l1
    out = pl.pallas_call(
        _stage3_kernel,
        grid=(NGRAPH,),
        in_specs=[_edge_spec(GROW), _graph_spec(l1), _graph_spec(d1),
                  _full_spec(l1, l1), _full_spec(8, l1),
                  _full_spec(l1, l1), _full_spec(8, l1),
                  _full_spec(d2, 2), _full_spec(8, 2)],
        out_specs=[pl.BlockSpec((1, 1, 2), lambda g: (g, 0, 0))],
        out_shape=[jax.ShapeDtypeStruct((NGRAPH, 1, 2), jnp.float32)],
        compiler_params=_CPARAMS,
    )(g1, c1, xb1, w2, bias2, w3, bias3, wc, bc)[0]
    return out.reshape(NGRAPH, 2)


def _make_sc_gather(v, d, b):
    per_w = b // _SC_WORKERS
    n_chunks = per_w // _GATHER_CHUNK
    mesh = plsc.VectorSubcoreMesh(core_axis_name="c", subcore_axis_name="s")

    @functools.partial(
        pl.kernel, mesh=mesh,
        out_type=jax.ShapeDtypeStruct((b, d), jnp.float32),
        scratch_types=[
            pltpu.VMEM((_GATHER_CHUNK,), jnp.int32),
            pltpu.VMEM((_GATHER_CHUNK, d), jnp.float32),
            pltpu.SemaphoreType.DMA,
        ],
    )
    def gather_kernel(table_hbm, idx_hbm, out_hbm, idx_v, rows_v, sem):
        wid = lax.axis_index("s") * _SC_CORES + lax.axis_index("c")
        base = wid * per_w

        @pl.loop(0, n_chunks)
        def _(ci):
            off = base + ci * _GATHER_CHUNK
            pltpu.sync_copy(idx_hbm.at[pl.ds(off, _GATHER_CHUNK)], idx_v)
            pltpu.async_copy(table_hbm.at[idx_v], rows_v, sem).wait()
            pltpu.sync_copy(rows_v, out_hbm.at[pl.ds(off, _GATHER_CHUNK)])

    return gather_kernel


def _gather_rows(table, idx):
    v, d = table.shape
    return _make_sc_gather(v, d, idx.shape[0])(table, idx)


def _fold_first_layer(p, feat):
    s = p['gamma'] / jnp.sqrt(p['rv'] + 1e-5)
    t = p['beta'] - p['rm'] * s
    wa = p['W'][:, :feat]
    wb = p['W'][:, feat:]
    w1b = jnp.pad((wb * s[:, None]).T, ((0, 0), (0, GROW - wb.shape[0])))
    w1c = ((wa - wb) * s[:, None]).T
    bias1 = jnp.tile((p['b'] * s + t)[None, :], (8, 1))
    return w1b, w1c, bias1


def _fold_layer(p):
    s = p['gamma'] / jnp.sqrt(p['rv'] + 1e-5)
    t = p['beta'] - p['rm'] * s
    return (p['W'] * s[:, None]).T, jnp.tile((p['b'] * s + t)[None, :], (8, 1))


def kernel(x, batch, params):
    del batch
    blk0, blk1 = params['blocks']
    w1b0, w1c0, bias1_0 = _fold_first_layer(blk0[0], FEAT)
    w2_0, bias2_0 = _fold_layer(blk0[1])
    w3_0, bias3_0 = _fold_layer(blk0[2])
    w1b1, w1c1, bias1_1 = _fold_first_layer(blk1[0], FEAT + w1c0.shape[1])
    w2_1, bias2_1 = _fold_layer(blk1[1])
    w3_1, bias3_1 = _fold_layer(blk1[2])
    wc = params['fc1']['W'].T @ params['fc2']['W'].T
    bc = jnp.tile((params['fc1']['b'] @ params['fc2']['W'].T
                   + params['fc2']['b'])[None, :], (8, 1))

    xp = jnp.pad(x.reshape(NGRAPH, NODES, FEAT),
                 ((0, 0), (0, NPAD - NODES), (0, 0))).reshape(NGRAPH * NPAD, FEAT)

    nbr0, b0, c0 = _stage1(xp, w1b0, w1c0, bias1_0)
    e0 = nbr0.transpose(0, 2, 1).reshape(-1)
    g0 = _gather_rows(b0, e0)
    nbr1, b1, c1, xb1 = _stage2(g0, c0, xp, w2_0, bias2_0, w3_0, bias3_0,
                                w1b1, w1c1, bias1_1)
    e1 = nbr1.transpose(0, 2, 1).reshape(-1)
    g1 = _gather_rows(b1, e1)
    return _stage3(g1, c1, xb1, w2_1, bias2_1, w3_1, bias3_1, wc, bc)


# speedup vs baseline: 3.0077x; 3.0077x over previous
import functools

import jax
import jax.numpy as jnp
from jax import lax
from jax.experimental import pallas as pl
from jax.experimental.pallas import tpu as pltpu
from jax.experimental.pallas import tpu_sc as plsc

NGRAPH = 20
NODES = 500
NPAD = 512
KNN = 16
FEAT = 128
BIG = 1e30
_HI = lax.Precision.HIGHEST

_SC_CORES = 2
_SC_SUBCORES = 16
_SC_WORKERS = _SC_CORES * _SC_SUBCORES
_GATHER_CHUNK = 128
GROW = 128


def _dot(a, b, prec=lax.Precision.DEFAULT):
    return lax.dot_general(a, b, (((1,), (0,)), ((), ())),
                           preferred_element_type=jnp.float32, precision=prec)


def _dot_t(a, b, prec=_HI):
    return lax.dot_general(a, b, (((1,), (1,)), ((), ())),
                           preferred_element_type=jnp.float32, precision=prec)


_TKROWS = 64


def _topk_neighbors(d2_ref, nbr_ref, base):
    col = lax.broadcasted_iota(jnp.int32, (_TKROWS, NPAD), 1)
    lane_k = lax.broadcasted_iota(jnp.int32, (_TKROWS, KNN), 1)

    for ci in range(NPAD // _TKROWS):
        d2 = d2_ref[ci * _TKROWS:(ci + 1) * _TKROWS, :]
        idx_acc = jnp.zeros((_TKROWS, KNN), jnp.int32)
        for t in range(KNN):
            m = jnp.min(d2, axis=1, keepdims=True)
            idx = jnp.min(jnp.where(d2 == m, col, jnp.int32(2 ** 30)),
                          axis=1, keepdims=True)
            idx_acc = jnp.where(lane_k == t, idx + base, idx_acc)
            d2 = jnp.where(col == idx, BIG, d2)
        nbr_ref[0, ci * _TKROWS:(ci + 1) * _TKROWS, :] = idx_acc


def _pair_dist(feat, d2_ref):
    g = _dot_t(feat, feat)
    sq = jnp.sum(feat * feat, axis=1, keepdims=True)
    ones = jnp.ones((NPAD, 1), jnp.float32)
    sqr = _dot_t(ones, sq)
    d2 = sqr - 2.0 * g
    col = lax.broadcasted_iota(jnp.int32, (NPAD, NPAD), 1)
    row = lax.broadcasted_iota(jnp.int32, (NPAD, NPAD), 0)
    d2_ref[...] = jnp.where(col >= NODES, BIG,
                            jnp.where(col == row, d2 + 1e9, d2))


def _stage1_kernel(x_ref, w1b_ref, w1c_ref, bias1_ref,
                   nbr_ref, b0_ref, c0_ref, d2_ref):
    g = pl.program_id(0)
    x = x_ref[...]
    _pair_dist(x[:, 0:2], d2_ref)
    _topk_neighbors(d2_ref, nbr_ref, g * NPAD)
    b0_ref[...] = _dot(x, w1b_ref[...]).astype(b0_ref.dtype)
    c0_ref[...] = _dot(x, w1c_ref[...]) + bias1_ref[0:1, :]


def _edge_mlp_tail(h1, w2_ref, bias2_ref, w3_ref, bias3_ref):
    h = jnp.maximum(h1, 0.0)
    h = jnp.maximum(_dot(h, w2_ref[...]) + bias2_ref[0:1, :], 0.0)
    h = jnp.maximum(_dot(h, w3_ref[...]) + bias3_ref[0:1, :], 0.0)
    acc = h[0:NPAD]
    for k in range(1, KNN):
        acc = acc + h[k * NPAD:(k + 1) * NPAD]
    return acc * (1.0 / KNN)


def _stage2_kernel(g0_ref, c0_ref, x_ref,
                   w2_ref, bias2_ref, w3_ref, bias3_ref,
                   w1b_ref, w1c_ref, bias1_ref,
                   nbr_ref, b1_ref, c1_ref, xb1_ref, d2_ref):
    g = pl.program_id(0)
    c0 = c0_ref[...]
    l0 = c0.shape[1]
    h1 = (g0_ref[:, 0:l0].astype(jnp.float32)
          + jnp.concatenate([c0] * KNN, axis=0))
    conv = _edge_mlp_tail(h1, w2_ref, bias2_ref, w3_ref, bias3_ref)
    xb1 = jnp.concatenate([conv, x_ref[...]], axis=1)
    xb1_ref[...] = xb1
    _pair_dist(xb1, d2_ref)
    _topk_neighbors(d2_ref, nbr_ref, g * NPAD)
    b1_ref[...] = _dot(xb1, w1b_ref[...]).astype(b1_ref.dtype)
    c1_ref[...] = _dot(xb1, w1c_ref[...]) + bias1_ref[0:1, :]


def _stage3_kernel(g1_ref, c1_ref, xb1_ref,
                   w2_ref, bias2_ref, w3_ref, bias3_ref,
                   wc_ref, bc_ref, out_ref):
    c1 = c1_ref[...]
    l1 = c1.shape[1]
    h1 = (g1_ref[:, 0:l1].astype(jnp.float32)
          + jnp.concatenate([c1] * KNN, axis=0))
    conv = _edge_mlp_tail(h1, w2_ref, bias2_ref, w3_ref, bias3_ref)
    xb2 = jnp.concatenate([conv, xb1_ref[...]], axis=1)
    rows = lax.broadcasted_iota(jnp.int32, (NPAD, 1), 0)
    xb2 = jnp.where(rows < NODES, xb2, 0.0)
    pooled = jnp.sum(xb2, axis=0, keepdims=True) * (1.0 / NODES)
    out_ref[0] = _dot(pooled, wc_ref[...]) + bc_ref[0:1, :]


def _graph_spec(cols):
    return pl.BlockSpec((NPAD, cols), lambda g: (g, 0))


def _edge_spec(cols):
    return pl.BlockSpec((NPAD * KNN, cols), lambda g: (g, 0))


def _full_spec(rows, cols):
    return pl.BlockSpec((rows, cols), lambda g: (0, 0))


_NBR_SPEC = pl.BlockSpec((1, NPAD, KNN), lambda g: (g, 0, 0))
_CPARAMS = pltpu.CompilerParams(dimension_semantics=("arbitrary",))


def _stage1(xp, w1b, w1c, bias1):
    l0 = w1c.shape[1]
    return pl.pallas_call(
        _stage1_kernel,
        grid=(NGRAPH,),
        in_specs=[_graph_spec(FEAT), _full_spec(FEAT, GROW),
                  _full_spec(FEAT, l0), _full_spec(8, l0)],
        out_specs=[_NBR_SPEC, _graph_spec(GROW), _graph_spec(l0)],
        out_shape=[
            jax.ShapeDtypeStruct((NGRAPH, NPAD, KNN), jnp.int32),
            jax.ShapeDtypeStruct((NGRAPH * NPAD, GROW), jnp.float32),
            jax.ShapeDtypeStruct((NGRAPH * NPAD, l0), jnp.float32),
        ],
        scratch_shapes=[pltpu.VMEM((NPAD, NPAD), jnp.float32)],
        compiler_params=_CPARAMS,
    )(xp, w1b, w1c, bias1)


def _stage2(g0, c0, xp, w2, bias2, w3, bias3, w1b, w1c, bias1):
    l0 = c0.shape[1]
    l1 = w1c.shape[1]
    d1 = FEAT + l0
    return pl.pallas_call(
        _stage2_kernel,
        grid=(NGRAPH,),
        in_specs=[_edge_spec(GROW), _graph_spec(l0), _graph_spec(FEAT),
                  _full_spec(l0, l0), _full_spec(8, l0),
                  _full_spec(l0, l0), _full_spec(8, l0),
                  _full_spec(d1, GROW), _full_spec(d1, l1), _full_spec(8, l1)],
        out_specs=[_NBR_SPEC, _graph_spec(GROW), _graph_spec(l1),
                   _graph_spec(d1)],
        out_shape=[
            jax.ShapeDtypeStruct((NGRAPH, NPAD, KNN), jnp.int32),
            jax.ShapeDtypeStruct((NGRAPH * NPAD, GROW), jnp.float32),
            jax.ShapeDtypeStruct((NGRAPH * NPAD, l1), jnp.float32),
            jax.ShapeDtypeStruct((NGRAPH * NPAD, d1), jnp.float32),
        ],
        scratch_shapes=[pltpu.VMEM((NPAD, NPAD), jnp.float32)],
        compiler_params=_CPARAMS,
    )(g0, c0, xp, w2, bias2, w3, bias3, w1b, w1c, bias1)


def _stage3(g1, c1, xb1, w2, bias2, w3, bias3, wc, bc):
    l1 = c1.shape[1]
    d1 = xb1.shape[1]
    d2 = d1 + l1
    out = pl.pallas_call(
        _stage3_kernel,
        grid=(NGRAPH,),
        in_specs=[_edge_spec(GROW), _graph_spec(l1), _graph_spec(d1),
                  _full_spec(l1, l1), _full_spec(8, l1),
                  _full_spec(l1, l1), _full_spec(8, l1),
                  _full_spec(d2, 2), _full_spec(8, 2)],
        out_specs=[pl.BlockSpec((1, 1, 2), lambda g: (g, 0, 0))],
        out_shape=[jax.ShapeDtypeStruct((NGRAPH, 1, 2), jnp.float32)],
        compiler_params=_CPARAMS,
    )(g1, c1, xb1, w2, bias2, w3, bias3, wc, bc)[0]
    return out.reshape(NGRAPH, 2)


def _make_sc_gather(v, d, b, dtype):
    per_w = b // _SC_WORKERS
    nb = per_w // _GATHER_CHUNK
    mesh = plsc.VectorSubcoreMesh(core_axis_name="c", subcore_axis_name="s")

    @functools.partial(
        pl.kernel, mesh=mesh,
        out_type=jax.ShapeDtypeStruct((b, d), dtype),
        scratch_types=[
            pltpu.VMEM((per_w,), jnp.int32),
            pltpu.VMEM((2, _GATHER_CHUNK, d), dtype),
            pltpu.SemaphoreType.DMA,
            pltpu.SemaphoreType.DMA((2,)),
            pltpu.SemaphoreType.DMA((2,)),
        ],
    )
    def gather_kernel(table_hbm, idx_hbm, out_hbm, idx_v, rows_v,
                      isem, gsem, osem):
        wid = lax.axis_index("s") * _SC_CORES + lax.axis_index("c")
        base = wid * per_w
        pltpu.make_async_copy(idx_hbm.at[pl.ds(base, per_w)], idx_v,
                              isem).start()

        def g_copy(c, buf):
            sl = idx_v.at[pl.ds(c * _GATHER_CHUNK, _GATHER_CHUNK)]
            return pltpu.make_async_copy(table_hbm.at[sl], rows_v.at[buf],
                                         gsem.at[buf])

        def o_copy(c, buf):
            dst = out_hbm.at[pl.ds(base + c * _GATHER_CHUNK, _GATHER_CHUNK)]
            return pltpu.make_async_copy(rows_v.at[buf], dst, osem.at[buf])

        pltpu.make_async_copy(idx_hbm.at[pl.ds(base, per_w)], idx_v,
                              isem).wait()
        g_copy(0, 0).start()

        @pl.loop(0, nb // 2)
        def _(i):
            for bsel in (0, 1):
                c = 2 * i + bsel

                @pl.when(c >= 1)
                def _():
                    o_copy(c - 1, 1 - bsel).wait()

                @pl.when(c + 1 < nb)
                def _():
                    g_copy(c + 1, 1 - bsel).start()

                g_copy(c, bsel).wait()
                o_copy(c, bsel).start()

        o_copy(nb - 1, (nb - 1) % 2).wait()

    return gather_kernel


def _gather_rows(table, idx):
    v, d = table.shape
    return _make_sc_gather(v, d, idx.shape[0], table.dtype)(table, idx)


def _fold_first_layer(p, feat):
    s = p['gamma'] / jnp.sqrt(p['rv'] + 1e-5)
    t = p['beta'] - p['rm'] * s
    wa = p['W'][:, :feat]
    wb = p['W'][:, feat:]
    w1b = jnp.pad((wb * s[:, None]).T, ((0, 0), (0, GROW - wb.shape[0])))
    w1c = ((wa - wb) * s[:, None]).T
    bias1 = jnp.tile((p['b'] * s + t)[None, :], (8, 1))
    return w1b, w1c, bias1


def _fold_layer(p):
    s = p['gamma'] / jnp.sqrt(p['rv'] + 1e-5)
    t = p['beta'] - p['rm'] * s
    return (p['W'] * s[:, None]).T, jnp.tile((p['b'] * s + t)[None, :], (8, 1))


def kernel(x, batch, params):
    del batch
    blk0, blk1 = params['blocks']
    w1b0, w1c0, bias1_0 = _fold_first_layer(blk0[0], FEAT)
    w2_0, bias2_0 = _fold_layer(blk0[1])
    w3_0, bias3_0 = _fold_layer(blk0[2])
    w1b1, w1c1, bias1_1 = _fold_first_layer(blk1[0], FEAT + w1c0.shape[1])
    w2_1, bias2_1 = _fold_layer(blk1[1])
    w3_1, bias3_1 = _fold_layer(blk1[2])
    wc = params['fc1']['W'].T @ params['fc2']['W'].T
    bc = jnp.tile((params['fc1']['b'] @ params['fc2']['W'].T
                   + params['fc2']['b'])[None, :], (8, 1))

    xp = jnp.pad(x.reshape(NGRAPH, NODES, FEAT),
                 ((0, 0), (0, NPAD - NODES), (0, 0))).reshape(NGRAPH * NPAD, FEAT)

    nbr0, b0, c0 = _stage1(xp, w1b0, w1c0, bias1_0)
    e0 = nbr0.transpose(0, 2, 1).reshape(-1)
    g0 = _gather_rows(b0, e0)
    nbr1, b1, c1, xb1 = _stage2(g0, c0, xp, w2_0, bias2_0, w3_0, bias3_0,
                                w1b1, w1c1, bias1_1)
    e1 = nbr1.transpose(0, 2, 1).reshape(-1)
    g1 = _gather_rows(b1, e1)
    return _stage3(g1, c1, xb1, w2_1, bias2_1, w3_1, bias3_1, wc, bc)


# speedup vs baseline: 3.6401x; 1.2103x over previous
import functools

import jax
import jax.numpy as jnp
from jax import lax
from jax.experimental import pallas as pl
from jax.experimental.pallas import tpu as pltpu
from jax.experimental.pallas import tpu_sc as plsc

NGRAPH = 20
NODES = 500
NPAD = 512
KNN = 16
FEAT = 128
BIG = 1e30
_HI = lax.Precision.HIGHEST

_SC_CORES = 2
_SC_SUBCORES = 16
_SC_WORKERS = _SC_CORES * _SC_SUBCORES
_GATHER_CHUNK = 128
GROW = 128


def _dot(a, b, prec=lax.Precision.DEFAULT):
    return lax.dot_general(a, b, (((1,), (0,)), ((), ())),
                           preferred_element_type=jnp.float32, precision=prec)


def _dot_t(a, b, prec=_HI):
    return lax.dot_general(a, b, (((1,), (1,)), ((), ())),
                           preferred_element_type=jnp.float32, precision=prec)


_TKROWS = 64


def _topk_neighbors(d2_ref, nbr_ref, base):
    col = lax.broadcasted_iota(jnp.int32, (_TKROWS, NPAD), 1)
    lane_k = lax.broadcasted_iota(jnp.int32, (_TKROWS, KNN), 1)

    for ci in range(NPAD // _TKROWS):
        d2 = d2_ref[ci * _TKROWS:(ci + 1) * _TKROWS, :]
        idx_acc = jnp.zeros((_TKROWS, KNN), jnp.int32)
        for t in range(KNN):
            m = jnp.min(d2, axis=1, keepdims=True)
            idx = jnp.min(jnp.where(d2 == m, col, jnp.int32(2 ** 30)),
                          axis=1, keepdims=True)
            idx_acc = jnp.where(lane_k == t, idx + base, idx_acc)
            d2 = jnp.where(col == idx, BIG, d2)
        nbr_ref[0, ci * _TKROWS:(ci + 1) * _TKROWS, :] = idx_acc


def _pair_dist(feat, d2_ref):
    g = _dot_t(feat, feat)
    sq = jnp.sum(feat * feat, axis=1, keepdims=True)
    ones = jnp.ones((NPAD, 1), jnp.float32)
    sqr = _dot_t(ones, sq)
    d2 = sqr - 2.0 * g
    col = lax.broadcasted_iota(jnp.int32, (NPAD, NPAD), 1)
    row = lax.broadcasted_iota(jnp.int32, (NPAD, NPAD), 0)
    d2_ref[...] = jnp.where(col >= NODES, BIG,
                            jnp.where(col == row, d2 + 1e9, d2))


def _stage1_kernel(x_ref, w1b_ref, w1c_ref, bias1_ref,
                   nbr_ref, b0_ref, c0_ref, d2_ref):
    g = pl.program_id(0)
    x = x_ref[...]
    _pair_dist(x[:, 0:2], d2_ref)
    _topk_neighbors(d2_ref, nbr_ref, g * NPAD)
    b0_ref[...] = _dot(x, w1b_ref[...]).astype(b0_ref.dtype)
    c0_ref[...] = _dot(x, w1c_ref[...]) + bias1_ref[0:1, :]


def _edge_mlp_tail(h1, w2_ref, bias2_ref, w3_ref, bias3_ref):
    h = jnp.maximum(h1, 0.0)
    h = jnp.maximum(_dot(h, w2_ref[...]) + bias2_ref[0:1, :], 0.0)
    h = jnp.maximum(_dot(h, w3_ref[...]) + bias3_ref[0:1, :], 0.0)
    acc = h[0:NPAD]
    for k in range(1, KNN):
        acc = acc + h[k * NPAD:(k + 1) * NPAD]
    return acc * (1.0 / KNN)


def _stage2_kernel(g0_ref, c0_ref, x_ref,
                   w2_ref, bias2_ref, w3_ref, bias3_ref,
                   w1b_ref, w1c_ref, bias1_ref,
                   nbr_ref, b1_ref, c1_ref, xb1_ref, d2_ref):
    g = pl.program_id(0)
    c0 = c0_ref[...]
    l0 = c0.shape[1]
    h1 = (g0_ref[:, 0:l0].astype(jnp.float32)
          + jnp.concatenate([c0] * KNN, axis=0))
    conv = _edge_mlp_tail(h1, w2_ref, bias2_ref, w3_ref, bias3_ref)
    xb1 = jnp.concatenate([conv, x_ref[...]], axis=1)
    xb1_ref[...] = xb1
    _pair_dist(xb1, d2_ref)
    _topk_neighbors(d2_ref, nbr_ref, g * NPAD)
    b1_ref[...] = _dot(xb1, w1b_ref[...]).astype(b1_ref.dtype)
    c1_ref[...] = _dot(xb1, w1c_ref[...]) + bias1_ref[0:1, :]


def _stage3_kernel(g1_ref, c1_ref, xb1_ref,
                   w2_ref, bias2_ref, w3_ref, bias3_ref,
                   wc_ref, bc_ref, out_ref):
    c1 = c1_ref[...]
    l1 = c1.shape[1]
    h1 = (g1_ref[:, 0:l1].astype(jnp.float32)
          + jnp.concatenate([c1] * KNN, axis=0))
    conv = _edge_mlp_tail(h1, w2_ref, bias2_ref, w3_ref, bias3_ref)
    xb2 = jnp.concatenate([conv, xb1_ref[...]], axis=1)
    rows = lax.broadcasted_iota(jnp.int32, (NPAD, 1), 0)
    xb2 = jnp.where(rows < NODES, xb2, 0.0)
    pooled = jnp.sum(xb2, axis=0, keepdims=True) * (1.0 / NODES)
    out_ref[0] = _dot(pooled, wc_ref[...]) + bc_ref[0:1, :]


def _graph_spec(cols):
    return pl.BlockSpec((NPAD, cols), lambda g: (g, 0))


def _edge_spec(cols):
    return pl.BlockSpec((NPAD * KNN, cols), lambda g: (g, 0))


def _full_spec(rows, cols):
    return pl.BlockSpec((rows, cols), lambda g: (0, 0))


_NBR_SPEC = pl.BlockSpec((1, NPAD, KNN), lambda g: (g, 0, 0))
_CPARAMS = pltpu.CompilerParams(dimension_semantics=("arbitrary",))


def _stage1(xp, w1b, w1c, bias1):
    l0 = w1c.shape[1]
    ng = xp.shape[0] // NPAD
    return pl.pallas_call(
        _stage1_kernel,
        grid=(ng,),
        in_specs=[_graph_spec(FEAT), _full_spec(FEAT, GROW),
                  _full_spec(FEAT, l0), _full_spec(8, l0)],
        out_specs=[_NBR_SPEC, _graph_spec(GROW), _graph_spec(l0)],
        out_shape=[
            jax.ShapeDtypeStruct((ng, NPAD, KNN), jnp.int32),
            jax.ShapeDtypeStruct((ng * NPAD, GROW), jnp.float32),
            jax.ShapeDtypeStruct((ng * NPAD, l0), jnp.float32),
        ],
        scratch_shapes=[pltpu.VMEM((NPAD, NPAD), jnp.float32)],
        compiler_params=_CPARAMS,
    )(xp, w1b, w1c, bias1)


def _stage2(g0, c0, xp, w2, bias2, w3, bias3, w1b, w1c, bias1):
    l0 = c0.shape[1]
    l1 = w1c.shape[1]
    d1 = FEAT + l0
    ng = c0.shape[0] // NPAD
    return pl.pallas_call(
        _stage2_kernel,
        grid=(ng,),
        in_specs=[_edge_spec(GROW), _graph_spec(l0), _graph_spec(FEAT),
                  _full_spec(l0, l0), _full_spec(8, l0),
                  _full_spec(l0, l0), _full_spec(8, l0),
                  _full_spec(d1, GROW), _full_spec(d1, l1), _full_spec(8, l1)],
        out_specs=[_NBR_SPEC, _graph_spec(GROW), _graph_spec(l1),
                   _graph_spec(d1)],
        out_shape=[
            jax.ShapeDtypeStruct((ng, NPAD, KNN), jnp.int32),
            jax.ShapeDtypeStruct((ng * NPAD, GROW), jnp.float32),
            jax.ShapeDtypeStruct((ng * NPAD, l1), jnp.float32),
            jax.ShapeDtypeStruct((ng * NPAD, d1), jnp.float32),
        ],
        scratch_shapes=[pltpu.VMEM((NPAD, NPAD), jnp.float32)],
        compiler_params=_CPARAMS,
    )(g0, c0, xp, w2, bias2, w3, bias3, w1b, w1c, bias1)


def _stage3(g1, c1, xb1, w2, bias2, w3, bias3, wc, bc):
    l1 = c1.shape[1]
    d1 = xb1.shape[1]
    d2 = d1 + l1
    ng = c1.shape[0] // NPAD
    out = pl.pallas_call(
        _stage3_kernel,
        grid=(ng,),
        in_specs=[_edge_spec(GROW), _graph_spec(l1), _graph_spec(d1),
                  _full_spec(l1, l1), _full_spec(8, l1),
                  _full_spec(l1, l1), _full_spec(8, l1),
                  _full_spec(d2, 2), _full_spec(8, 2)],
        out_specs=[pl.BlockSpec((1, 1, 2), lambda g: (g, 0, 0))],
        out_shape=[jax.ShapeDtypeStruct((ng, 1, 2), jnp.float32)],
        compiler_params=_CPARAMS,
    )(g1, c1, xb1, w2, bias2, w3, bias3, wc, bc)[0]
    return out.reshape(ng, 2)


def _make_sc_gather(v, d, b, dtype):
    per_w = b // _SC_WORKERS
    nb = per_w // _GATHER_CHUNK
    mesh = plsc.VectorSubcoreMesh(core_axis_name="c", subcore_axis_name="s")

    @functools.partial(
        pl.kernel, mesh=mesh,
        out_type=jax.ShapeDtypeStruct((b, d), dtype),
        scratch_types=[
            pltpu.VMEM((per_w,), jnp.int32),
            pltpu.VMEM((2, _GATHER_CHUNK, d), dtype),
            pltpu.SemaphoreType.DMA,
            pltpu.SemaphoreType.DMA((2,)),
            pltpu.SemaphoreType.DMA((2,)),
        ],
    )
    def gather_kernel(table_hbm, idx_hbm, out_hbm, idx_v, rows_v,
                      isem, gsem, osem):
        wid = lax.axis_index("s") * _SC_CORES + lax.axis_index("c")
        base = wid * per_w
        pltpu.make_async_copy(idx_hbm.at[pl.ds(base, per_w)], idx_v,
                              isem).start()

        def g_copy(c, buf):
            sl = idx_v.at[pl.ds(c * _GATHER_CHUNK, _GATHER_CHUNK)]
            return pltpu.make_async_copy(table_hbm.at[sl], rows_v.at[buf],
                                         gsem.at[buf])

        def o_copy(c, buf):
            dst = out_hbm.at[pl.ds(base + c * _GATHER_CHUNK, _GATHER_CHUNK)]
            return pltpu.make_async_copy(rows_v.at[buf], dst, osem.at[buf])

        pltpu.make_async_copy(idx_hbm.at[pl.ds(base, per_w)], idx_v,
                              isem).wait()
        g_copy(0, 0).start()

        @pl.loop(0, nb // 2)
        def _(i):
            for bsel in (0, 1):
                c = 2 * i + bsel

                @pl.when(c >= 1)
                def _():
                    o_copy(c - 1, 1 - bsel).wait()

                @pl.when(c + 1 < nb)
                def _():
                    g_copy(c + 1, 1 - bsel).start()

                g_copy(c, bsel).wait()
                o_copy(c, bsel).start()

        o_copy(nb - 1, (nb - 1) % 2).wait()

    return gather_kernel


def _gather_rows(table, idx):
    v, d = table.shape
    return _make_sc_gather(v, d, idx.shape[0], table.dtype)(table, idx)


def _fold_first_layer(p, feat):
    s = p['gamma'] / jnp.sqrt(p['rv'] + 1e-5)
    t = p['beta'] - p['rm'] * s
    wa = p['W'][:, :feat]
    wb = p['W'][:, feat:]
    w1b = jnp.pad((wb * s[:, None]).T, ((0, 0), (0, GROW - wb.shape[0])))
    w1c = ((wa - wb) * s[:, None]).T
    bias1 = jnp.tile((p['b'] * s + t)[None, :], (8, 1))
    return w1b, w1c, bias1


def _fold_layer(p):
    s = p['gamma'] / jnp.sqrt(p['rv'] + 1e-5)
    t = p['beta'] - p['rm'] * s
    return (p['W'] * s[:, None]).T, jnp.tile((p['b'] * s + t)[None, :], (8, 1))


def kernel(x, batch, params):
    del batch
    blk0, blk1 = params['blocks']
    w1b0, w1c0, bias1_0 = _fold_first_layer(blk0[0], FEAT)
    w2_0, bias2_0 = _fold_layer(blk0[1])
    w3_0, bias3_0 = _fold_layer(blk0[2])
    w1b1, w1c1, bias1_1 = _fold_first_layer(blk1[0], FEAT + w1c0.shape[1])
    w2_1, bias2_1 = _fold_layer(blk1[1])
    w3_1, bias3_1 = _fold_layer(blk1[2])
    wc = params['fc1']['W'].T @ params['fc2']['W'].T
    bc = jnp.tile((params['fc1']['b'] @ params['fc2']['W'].T
                   + params['fc2']['b'])[None, :], (8, 1))

    xp = jnp.pad(x.reshape(NGRAPH, NODES, FEAT),
                 ((0, 0), (0, NPAD - NODES), (0, 0))).reshape(NGRAPH * NPAD, FEAT)

    half = NGRAPH // 2
    outs = []
    for xh in (xp[:half * NPAD], xp[half * NPAD:]):
        nbr0, b0, c0 = _stage1(xh, w1b0, w1c0, bias1_0)
        e0 = nbr0.transpose(0, 2, 1).reshape(-1)
        g0 = _gather_rows(b0, e0)
        nbr1, b1, c1, xb1 = _stage2(g0, c0, xh, w2_0, bias2_0, w3_0, bias3_0,
                                    w1b1, w1c1, bias1_1)
        e1 = nbr1.transpose(0, 2, 1).reshape(-1)
        g1 = _gather_rows(b1, e1)
        outs.append(_stage3(g1, c1, xb1, w2_1, bias2_1, w3_1, bias3_1, wc, bc))
    return jnp.concatenate(outs, axis=0)


# speedup vs baseline: 3.7589x; 1.0326x over previous
import functools

import jax
import jax.numpy as jnp
from jax import lax
from jax.experimental import pallas as pl
from jax.experimental.pallas import tpu as pltpu
from jax.experimental.pallas import tpu_sc as plsc

NGRAPH = 20
NODES = 500
NPAD = 512
KNN = 16
FEAT = 128
BIG = 1e30
_HI = lax.Precision.HIGHEST

_SC_CORES = 2
_SC_SUBCORES = 16
_SC_WORKERS = _SC_CORES * _SC_SUBCORES
_GATHER_CHUNK = 128
GROW = 128


def _dot(a, b, prec=lax.Precision.DEFAULT):
    return lax.dot_general(a, b, (((1,), (0,)), ((), ())),
                           preferred_element_type=jnp.float32, precision=prec)


def _dot_t(a, b, prec=_HI):
    return lax.dot_general(a, b, (((1,), (1,)), ((), ())),
                           preferred_element_type=jnp.float32, precision=prec)


_TKROWS = 64


def _topk_neighbors(d2_ref, nbr_ref, base):
    col = lax.broadcasted_iota(jnp.int32, (_TKROWS, NPAD), 1)
    lane_k = lax.broadcasted_iota(jnp.int32, (_TKROWS, KNN), 1)

    for ci in range(NPAD // _TKROWS):
        d2 = d2_ref[ci * _TKROWS:(ci + 1) * _TKROWS, :]
        idx_acc = jnp.zeros((_TKROWS, KNN), jnp.int32)
        for t in range(KNN):
            m = jnp.min(d2, axis=1, keepdims=True)
            idx = jnp.min(jnp.where(d2 == m, col, jnp.int32(2 ** 30)),
                          axis=1, keepdims=True)
            idx_acc = jnp.where(lane_k == t, idx + base, idx_acc)
            d2 = jnp.where(col == idx, BIG, d2)
        nbr_ref[0, ci * _TKROWS:(ci + 1) * _TKROWS, :] = idx_acc


def _pair_dist(feat, d2_ref):
    hi = feat.astype(jnp.bfloat16)
    lo = (feat - hi.astype(jnp.float32)).astype(jnp.bfloat16)
    g = (_dot_t(hi, hi, prec=lax.Precision.DEFAULT)
         + (_dot_t(hi, lo, prec=lax.Precision.DEFAULT)
            + _dot_t(lo, hi, prec=lax.Precision.DEFAULT)))
    sq = jnp.sum(feat * feat, axis=1, keepdims=True)
    ones = jnp.ones((NPAD, 1), jnp.float32)
    sqr = _dot_t(ones, sq)
    d2 = sqr - 2.0 * g
    col = lax.broadcasted_iota(jnp.int32, (NPAD, NPAD), 1)
    row = lax.broadcasted_iota(jnp.int32, (NPAD, NPAD), 0)
    d2_ref[...] = jnp.where(col >= NODES, BIG,
                            jnp.where(col == row, d2 + 1e9, d2))


def _stage1_kernel(x_ref, w1b_ref, w1c_ref, bias1_ref,
                   nbr_ref, b0_ref, c0_ref, d2_ref):
    g = pl.program_id(0)
    x = x_ref[...]
    _pair_dist(x[:, 0:2], d2_ref)
    _topk_neighbors(d2_ref, nbr_ref, g * NPAD)
    b0_ref[...] = _dot(x, w1b_ref[...]).astype(b0_ref.dtype)
    c0_ref[...] = _dot(x, w1c_ref[...]) + bias1_ref[0:1, :]


def _edge_mlp_tail(h1, w2_ref, bias2_ref, w3_ref, bias3_ref):
    h = jnp.maximum(h1, 0.0)
    h = jnp.maximum(_dot(h, w2_ref[...]) + bias2_ref[0:1, :], 0.0)
    h = jnp.maximum(_dot(h, w3_ref[...]) + bias3_ref[0:1, :], 0.0)
    acc = h[0:NPAD]
    for k in range(1, KNN):
        acc = acc + h[k * NPAD:(k + 1) * NPAD]
    return acc * (1.0 / KNN)


def _stage2_kernel(g0_ref, c0_ref, x_ref,
                   w2_ref, bias2_ref, w3_ref, bias3_ref,
                   w1b_ref, w1c_ref, bias1_ref,
                   nbr_ref, b1_ref, c1_ref, xb1_ref, d2_ref):
    g = pl.program_id(0)
    c0 = c0_ref[...]
    l0 = c0.shape[1]
    h1 = (g0_ref[:, 0:l0].astype(jnp.float32)
          + jnp.concatenate([c0] * KNN, axis=0))
    conv = _edge_mlp_tail(h1, w2_ref, bias2_ref, w3_ref, bias3_ref)
    xb1 = jnp.concatenate([conv, x_ref[...]], axis=1)
    xb1_ref[...] = xb1
    _pair_dist(xb1, d2_ref)
    _topk_neighbors(d2_ref, nbr_ref, g * NPAD)
    b1_ref[...] = _dot(xb1, w1b_ref[...]).astype(b1_ref.dtype)
    c1_ref[...] = _dot(xb1, w1c_ref[...]) + bias1_ref[0:1, :]


def _stage3_kernel(g1_ref, c1_ref, xb1_ref,
                   w2_ref, bias2_ref, w3_ref, bias3_ref,
                   wc_ref, bc_ref, out_ref):
    c1 = c1_ref[...]
    l1 = c1.shape[1]
    h1 = (g1_ref[:, 0:l1].astype(jnp.float32)
          + jnp.concatenate([c1] * KNN, axis=0))
    conv = _edge_mlp_tail(h1, w2_ref, bias2_ref, w3_ref, bias3_ref)
    xb2 = jnp.concatenate([conv, xb1_ref[...]], axis=1)
    rows = lax.broadcasted_iota(jnp.int32, (NPAD, 1), 0)
    xb2 = jnp.where(rows < NODES, xb2, 0.0)
    pooled = jnp.sum(xb2, axis=0, keepdims=True) * (1.0 / NODES)
    out_ref[0] = _dot(pooled, wc_ref[...]) + bc_ref[0:1, :]


def _graph_spec(cols):
    return pl.BlockSpec((NPAD, cols), lambda g: (g, 0))


def _edge_spec(cols):
    return pl.BlockSpec((NPAD * KNN, cols), lambda g: (g, 0))


def _full_spec(rows, cols):
    return pl.BlockSpec((rows, cols), lambda g: (0, 0))


_NBR_SPEC = pl.BlockSpec((1, NPAD, KNN), lambda g: (g, 0, 0))
_CPARAMS = pltpu.CompilerParams(dimension_semantics=("arbitrary",))


def _stage1(xp, w1b, w1c, bias1):
    l0 = w1c.shape[1]
    ng = xp.shape[0] // NPAD
    return pl.pallas_call(
        _stage1_kernel,
        grid=(ng,),
        in_specs=[_graph_spec(FEAT), _full_spec(FEAT, GROW),
                  _full_spec(FEAT, l0), _full_spec(8, l0)],
        out_specs=[_NBR_SPEC, _graph_spec(GROW), _graph_spec(l0)],
        out_shape=[
            jax.ShapeDtypeStruct((ng, NPAD, KNN), jnp.int32),
            jax.ShapeDtypeStruct((ng * NPAD, GROW), jnp.float32),
            jax.ShapeDtypeStruct((ng * NPAD, l0), jnp.float32),
        ],
        scratch_shapes=[pltpu.VMEM((NPAD, NPAD), jnp.float32)],
        compiler_params=_CPARAMS,
    )(xp, w1b, w1c, bias1)


def _stage2(g0, c0, xp, w2, bias2, w3, bias3, w1b, w1c, bias1):
    l0 = c0.shape[1]
    l1 = w1c.shape[1]
    d1 = FEAT + l0
    ng = c0.shape[0] // NPAD
    return pl.pallas_call(
        _stage2_kernel,
        grid=(ng,),
        in_specs=[_edge_spec(GROW), _graph_spec(l0), _graph_spec(FEAT),
                  _full_spec(l0, l0), _full_spec(8, l0),
                  _full_spec(l0, l0), _full_spec(8, l0),
                  _full_spec(d1, GROW), _full_spec(d1, l1), _full_spec(8, l1)],
        out_specs=[_NBR_SPEC, _graph_spec(GROW), _graph_spec(l1),
                   _graph_spec(d1)],
        out_shape=[
            jax.ShapeDtypeStruct((ng, NPAD, KNN), jnp.int32),
            jax.ShapeDtypeStruct((ng * NPAD, GROW), jnp.float32),
            jax.ShapeDtypeStruct((ng * NPAD, l1), jnp.float32),
            jax.ShapeDtypeStruct((ng * NPAD, d1), jnp.float32),
        ],
        scratch_shapes=[pltpu.VMEM((NPAD, NPAD), jnp.float32)],
        compiler_params=_CPARAMS,
    )(g0, c0, xp, w2, bias2, w3, bias3, w1b, w1c, bias1)


def _stage3(g1, c1, xb1, w2, bias2, w3, bias3, wc, bc):
    l1 = c1.shape[1]
    d1 = xb1.shape[1]
    d2 = d1 + l1
    ng = c1.shape[0] // NPAD
    out = pl.pallas_call(
        _stage3_kernel,
        grid=(ng,),
        in_specs=[_edge_spec(GROW), _graph_spec(l1), _graph_spec(d1),
                  _full_spec(l1, l1), _full_spec(8, l1),
                  _full_spec(l1, l1), _full_spec(8, l1),
                  _full_spec(d2, 2), _full_spec(8, 2)],
        out_specs=[pl.BlockSpec((1, 1, 2), lambda g: (g, 0, 0))],
        out_shape=[jax.ShapeDtypeStruct((ng, 1, 2), jnp.float32)],
        compiler_params=_CPARAMS,
    )(g1, c1, xb1, w2, bias2, w3, bias3, wc, bc)[0]
    return out.reshape(ng, 2)


def _make_sc_gather(v, d, b, dtype):
    per_w = b // _SC_WORKERS
    nb = per_w // _GATHER_CHUNK
    mesh = plsc.VectorSubcoreMesh(core_axis_name="c", subcore_axis_name="s")

    @functools.partial(
        pl.kernel, mesh=mesh,
        out_type=jax.ShapeDtypeStruct((b, d), dtype),
        scratch_types=[
            pltpu.VMEM((per_w,), jnp.int32),
            pltpu.VMEM((2, _GATHER_CHUNK, d), dtype),
            pltpu.SemaphoreType.DMA,
            pltpu.SemaphoreType.DMA((2,)),
            pltpu.SemaphoreType.DMA((2,)),
        ],
    )
    def gather_kernel(table_hbm, idx_hbm, out_hbm, idx_v, rows_v,
                      isem, gsem, osem):
        wid = lax.axis_index("s") * _SC_CORES + lax.axis_index("c")
        base = wid * per_w
        pltpu.make_async_copy(idx_hbm.at[pl.ds(base, per_w)], idx_v,
                              isem).start()

        def g_copy(c, buf):
            sl = idx_v.at[pl.ds(c * _GATHER_CHUNK, _GATHER_CHUNK)]
            return pltpu.make_async_copy(table_hbm.at[sl], rows_v.at[buf],
                                         gsem.at[buf])

        def o_copy(c, buf):
            dst = out_hbm.at[pl.ds(base + c * _GATHER_CHUNK, _GATHER_CHUNK)]
            return pltpu.make_async_copy(rows_v.at[buf], dst, osem.at[buf])

        pltpu.make_async_copy(idx_hbm.at[pl.ds(base, per_w)], idx_v,
                              isem).wait()
        g_copy(0, 0).start()

        @pl.loop(0, nb // 2)
        def _(i):
            for bsel in (0, 1):
                c = 2 * i + bsel

                @pl.when(c >= 1)
                def _():
                    o_copy(c - 1, 1 - bsel).wait()

                @pl.when(c + 1 < nb)
                def _():
                    g_copy(c + 1, 1 - bsel).start()

                g_copy(c, bsel).wait()
                o_copy(c, bsel).start()

        o_copy(nb - 1, (nb - 1) % 2).wait()

    return gather_kernel


def _gather_rows(table, idx):
    v, d = table.shape
    return _make_sc_gather(v, d, idx.shape[0], table.dtype)(table, idx)


def _fold_first_layer(p, feat):
    s = p['gamma'] / jnp.sqrt(p['rv'] + 1e-5)
    t = p['beta'] - p['rm'] * s
    wa = p['W'][:, :feat]
    wb = p['W'][:, feat:]
    w1b = jnp.pad((wb * s[:, None]).T, ((0, 0), (0, GROW - wb.shape[0])))
    w1c = ((wa - wb) * s[:, None]).T
    bias1 = jnp.tile((p['b'] * s + t)[None, :], (8, 1))
    return w1b, w1c, bias1


def _fold_layer(p):
    s = p['gamma'] / jnp.sqrt(p['rv'] + 1e-5)
    t = p['beta'] - p['rm'] * s
    return (p['W'] * s[:, None]).T, jnp.tile((p['b'] * s + t)[None, :], (8, 1))


def kernel(x, batch, params):
    del batch
    blk0, blk1 = params['blocks']
    w1b0, w1c0, bias1_0 = _fold_first_layer(blk0[0], FEAT)
    w2_0, bias2_0 = _fold_layer(blk0[1])
    w3_0, bias3_0 = _fold_layer(blk0[2])
    w1b1, w1c1, bias1_1 = _fold_first_layer(blk1[0], FEAT + w1c0.shape[1])
    w2_1, bias2_1 = _fold_layer(blk1[1])
    w3_1, bias3_1 = _fold_layer(blk1[2])
    wc = params['fc1']['W'].T @ params['fc2']['W'].T
    bc = jnp.tile((params['fc1']['b'] @ params['fc2']['W'].T
                   + params['fc2']['b'])[None, :], (8, 1))

    xp = jnp.pad(x.reshape(NGRAPH, NODES, FEAT),
                 ((0, 0), (0, NPAD - NODES), (0, 0))).reshape(NGRAPH * NPAD, FEAT)

    half = NGRAPH // 2
    outs = []
    for xh in (xp[:half * NPAD], xp[half * NPAD:]):
        nbr0, b0, c0 = _stage1(xh, w1b0, w1c0, bias1_0)
        e0 = nbr0.transpose(0, 2, 1).reshape(-1)
        g0 = _gather_rows(b0, e0)
        nbr1, b1, c1, xb1 = _stage2(g0, c0, xh, w2_0, bias2_0, w3_0, bias3_0,
                                    w1b1, w1c1, bias1_1)
        e1 = nbr1.transpose(0, 2, 1).reshape(-1)
        g1 = _gather_rows(b1, e1)
        outs.append(_stage3(g1, c1, xb1, w2_1, bias2_1, w3_1, bias3_1, wc, bc))
    return jnp.concatenate(outs, axis=0)


# speedup vs baseline: 4.3551x; 1.1586x over previous
import functools

import jax
import jax.numpy as jnp
from jax import lax
from jax.experimental import pallas as pl
from jax.experimental.pallas import tpu as pltpu
from jax.experimental.pallas import tpu_sc as plsc

NGRAPH = 20
NODES = 500
NPAD = 512
KNN = 16
FEAT = 128
BIG = 1e30
_HI = lax.Precision.HIGHEST

_SC_CORES = 2
_SC_SUBCORES = 16
_SC_WORKERS = _SC_CORES * _SC_SUBCORES
_GATHER_CHUNK = 128
GROW = 128


def _dot(a, b, prec=lax.Precision.DEFAULT):
    return lax.dot_general(a, b, (((1,), (0,)), ((), ())),
                           preferred_element_type=jnp.float32, precision=prec)


def _dot_t(a, b, prec=_HI):
    return lax.dot_general(a, b, (((1,), (1,)), ((), ())),
                           preferred_element_type=jnp.float32, precision=prec)


_TKROWS = 64


def _topk_neighbors(d2_ref, nbr_ref, base):
    lane_k = lax.broadcasted_iota(jnp.int32, (_TKROWS, KNN), 1)
    int_max = jnp.int32(2 ** 31 - 1)

    for ci in range(NPAD // _TKROWS):
        key = d2_ref[ci * _TKROWS:(ci + 1) * _TKROWS, :]
        macc = jnp.zeros((_TKROWS, KNN), jnp.int32)
        for t in range(KNN):
            m = jnp.min(key, axis=1, keepdims=True)
            macc = jnp.where(lane_k == t, m, macc)
            key = jnp.where(key == m, int_max, key)
        nbr_ref[0, ci * _TKROWS:(ci + 1) * _TKROWS, :] = (macc & 511) + base


def _pair_dist(feat, d2_ref):
    hi = feat.astype(jnp.bfloat16)
    lo = (feat - hi.astype(jnp.float32)).astype(jnp.bfloat16)
    g = (_dot_t(hi, hi, prec=lax.Precision.DEFAULT)
         + (_dot_t(hi, lo, prec=lax.Precision.DEFAULT)
            + _dot_t(lo, hi, prec=lax.Precision.DEFAULT)))
    sq = jnp.sum(feat * feat, axis=1, keepdims=True)
    ones = jnp.ones((NPAD, 1), jnp.float32)
    sqr = _dot_t(ones, sq)
    d2 = jnp.maximum(sq + sqr - 2.0 * g, 0.0)
    col = lax.broadcasted_iota(jnp.int32, (NPAD, NPAD), 1)
    row = lax.broadcasted_iota(jnp.int32, (NPAD, NPAD), 0)
    d2 = jnp.where(col >= NODES, BIG, jnp.where(col == row, d2 + 1e9, d2))
    bits = lax.bitcast_convert_type(d2, jnp.int32)
    d2_ref[...] = (bits & jnp.int32(-512)) | col


def _stage1_kernel(x_ref, w1b_ref, w1c_ref, bias1_ref,
                   nbr_ref, b0_ref, c0_ref, d2_ref):
    g = pl.program_id(0)
    x = x_ref[...]
    _pair_dist(x[:, 0:2], d2_ref)
    _topk_neighbors(d2_ref, nbr_ref, g * NPAD)
    b0_ref[...] = _dot(x, w1b_ref[...]).astype(b0_ref.dtype)
    c0_ref[...] = _dot(x, w1c_ref[...]) + bias1_ref[0:1, :]


def _edge_mlp_tail(h1, w2_ref, bias2_ref, w3_ref, bias3_ref):
    h = jnp.maximum(h1, 0.0)
    h = jnp.maximum(_dot(h, w2_ref[...]) + bias2_ref[0:1, :], 0.0)
    h = jnp.maximum(_dot(h, w3_ref[...]) + bias3_ref[0:1, :], 0.0)
    acc = h[0:NPAD]
    for k in range(1, KNN):
        acc = acc + h[k * NPAD:(k + 1) * NPAD]
    return acc * (1.0 / KNN)


def _stage2_kernel(g0_ref, c0_ref, x_ref,
                   w2_ref, bias2_ref, w3_ref, bias3_ref,
                   w1b_ref, w1c_ref, bias1_ref,
                   nbr_ref, b1_ref, c1_ref, xb1_ref, d2_ref):
    g = pl.program_id(0)
    c0 = c0_ref[...]
    l0 = c0.shape[1]
    h1 = (g0_ref[:, 0:l0].astype(jnp.float32)
          + jnp.concatenate([c0] * KNN, axis=0))
    conv = _edge_mlp_tail(h1, w2_ref, bias2_ref, w3_ref, bias3_ref)
    xb1 = jnp.concatenate([conv, x_ref[...]], axis=1)
    xb1_ref[...] = xb1
    _pair_dist(xb1, d2_ref)
    _topk_neighbors(d2_ref, nbr_ref, g * NPAD)
    b1_ref[...] = _dot(xb1, w1b_ref[...]).astype(b1_ref.dtype)
    c1_ref[...] = _dot(xb1, w1c_ref[...]) + bias1_ref[0:1, :]


def _stage3_kernel(g1_ref, c1_ref, xb1_ref,
                   w2_ref, bias2_ref, w3_ref, bias3_ref,
                   wc_ref, bc_ref, out_ref):
    c1 = c1_ref[...]
    l1 = c1.shape[1]
    h1 = (g1_ref[:, 0:l1].astype(jnp.float32)
          + jnp.concatenate([c1] * KNN, axis=0))
    conv = _edge_mlp_tail(h1, w2_ref, bias2_ref, w3_ref, bias3_ref)
    xb2 = jnp.concatenate([conv, xb1_ref[...]], axis=1)
    rows = lax.broadcasted_iota(jnp.int32, (NPAD, 1), 0)
    xb2 = jnp.where(rows < NODES, xb2, 0.0)
    pooled = jnp.sum(xb2, axis=0, keepdims=True) * (1.0 / NODES)
    out_ref[0] = _dot(pooled, wc_ref[...]) + bc_ref[0:1, :]


def _graph_spec(cols):
    return pl.BlockSpec((NPAD, cols), lambda g: (g, 0))


def _edge_spec(cols):
    return pl.BlockSpec((NPAD * KNN, cols), lambda g: (g, 0))


def _full_spec(rows, cols):
    return pl.BlockSpec((rows, cols), lambda g: (0, 0))


_NBR_SPEC = pl.BlockSpec((1, NPAD, KNN), lambda g: (g, 0, 0))
_CPARAMS = pltpu.CompilerParams(dimension_semantics=("arbitrary",))


def _stage1(xp, w1b, w1c, bias1):
    l0 = w1c.shape[1]
    ng = xp.shape[0] // NPAD
    return pl.pallas_call(
        _stage1_kernel,
        grid=(ng,),
        in_specs=[_graph_spec(FEAT), _full_spec(FEAT, GROW),
                  _full_spec(FEAT, l0), _full_spec(8, l0)],
        out_specs=[_NBR_SPEC, _graph_spec(GROW), _graph_spec(l0)],
        out_shape=[
            jax.ShapeDtypeStruct((ng, NPAD, KNN), jnp.int32),
            jax.ShapeDtypeStruct((ng * NPAD, GROW), jnp.float32),
            jax.ShapeDtypeStruct((ng * NPAD, l0), jnp.float32),
        ],
        scratch_shapes=[pltpu.VMEM((NPAD, NPAD), jnp.int32)],
        compiler_params=_CPARAMS,
    )(xp, w1b, w1c, bias1)


def _stage2(g0, c0, xp, w2, bias2, w3, bias3, w1b, w1c, bias1):
    l0 = c0.shape[1]
    l1 = w1c.shape[1]
    d1 = FEAT + l0
    ng = c0.shape[0] // NPAD
    return pl.pallas_call(
        _stage2_kernel,
        grid=(ng,),
        in_specs=[_edge_spec(GROW), _graph_spec(l0), _graph_spec(FEAT),
                  _full_spec(l0, l0), _full_spec(8, l0),
                  _full_spec(l0, l0), _full_spec(8, l0),
                  _full_spec(d1, GROW), _full_spec(d1, l1), _full_spec(8, l1)],
        out_specs=[_NBR_SPEC, _graph_spec(GROW), _graph_spec(l1),
                   _graph_spec(d1)],
        out_shape=[
            jax.ShapeDtypeStruct((ng, NPAD, KNN), jnp.int32),
            jax.ShapeDtypeStruct((ng * NPAD, GROW), jnp.float32),
            jax.ShapeDtypeStruct((ng * NPAD, l1), jnp.float32),
            jax.ShapeDtypeStruct((ng * NPAD, d1), jnp.float32),
        ],
        scratch_shapes=[pltpu.VMEM((NPAD, NPAD), jnp.int32)],
        compiler_params=_CPARAMS,
    )(g0, c0, xp, w2, bias2, w3, bias3, w1b, w1c, bias1)


def _stage3(g1, c1, xb1, w2, bias2, w3, bias3, wc, bc):
    l1 = c1.shape[1]
    d1 = xb1.shape[1]
    d2 = d1 + l1
    ng = c1.shape[0] // NPAD
    out = pl.pallas_call(
        _stage3_kernel,
        grid=(ng,),
        in_specs=[_edge_spec(GROW), _graph_spec(l1), _graph_spec(d1),
                  _full_spec(l1, l1), _full_spec(8, l1),
                  _full_spec(l1, l1), _full_spec(8, l1),
                  _full_spec(d2, 2), _full_spec(8, 2)],
        out_specs=[pl.BlockSpec((1, 1, 2), lambda g: (g, 0, 0))],
        out_shape=[jax.ShapeDtypeStruct((ng, 1, 2), jnp.float32)],
        compiler_params=_CPARAMS,
    )(g1, c1, xb1, w2, bias2, w3, bias3, wc, bc)[0]
    return out.reshape(ng, 2)


def _make_sc_gather(v, d, b, dtype):
    per_w = b // _SC_WORKERS
    nb = per_w // _GATHER_CHUNK
    mesh = plsc.VectorSubcoreMesh(core_axis_name="c", subcore_axis_name="s")

    @functools.partial(
        pl.kernel, mesh=mesh,
        out_type=jax.ShapeDtypeStruct((b, d), dtype),
        scratch_types=[
            pltpu.VMEM((per_w,), jnp.int32),
            pltpu.VMEM((2, _GATHER_CHUNK, d), dtype),
            pltpu.SemaphoreType.DMA,
            pltpu.SemaphoreType.DMA((2,)),
            pltpu.SemaphoreType.DMA((2,)),
        ],
    )
    def gather_kernel(table_hbm, idx_hbm, out_hbm, idx_v, rows_v,
                      isem, gsem, osem):
        wid = lax.axis_index("s") * _SC_CORES + lax.axis_index("c")
        base = wid * per_w
        pltpu.make_async_copy(idx_hbm.at[pl.ds(base, per_w)], idx_v,
                              isem).start()

        def g_copy(c, buf):
            sl = idx_v.at[pl.ds(c * _GATHER_CHUNK, _GATHER_CHUNK)]
            return pltpu.make_async_copy(table_hbm.at[sl], rows_v.at[buf],
                                         gsem.at[buf])

        def o_copy(c, buf):
            dst = out_hbm.at[pl.ds(base + c * _GATHER_CHUNK, _GATHER_CHUNK)]
            return pltpu.make_async_copy(rows_v.at[buf], dst, osem.at[buf])

        pltpu.make_async_copy(idx_hbm.at[pl.ds(base, per_w)], idx_v,
                              isem).wait()
        g_copy(0, 0).start()

        @pl.loop(0, nb // 2)
        def _(i):
            for bsel in (0, 1):
                c = 2 * i + bsel

                @pl.when(c >= 1)
                def _():
                    o_copy(c - 1, 1 - bsel).wait()

                @pl.when(c + 1 < nb)
                def _():
                    g_copy(c + 1, 1 - bsel).start()

                g_copy(c, bsel).wait()
                o_copy(c, bsel).start()

        o_copy(nb - 1, (nb - 1) % 2).wait()

    return gather_kernel


def _gather_rows(table, idx):
    v, d = table.shape
    return _make_sc_gather(v, d, idx.shape[0], table.dtype)(table, idx)


def _fold_first_layer(p, feat):
    s = p['gamma'] / jnp.sqrt(p['rv'] + 1e-5)
    t = p['beta'] - p['rm'] * s
    wa = p['W'][:, :feat]
    wb = p['W'][:, feat:]
    w1b = jnp.pad((wb * s[:, None]).T, ((0, 0), (0, GROW - wb.shape[0])))
    w1c = ((wa - wb) * s[:, None]).T
    bias1 = jnp.tile((p['b'] * s + t)[None, :], (8, 1))
    return w1b, w1c, bias1


def _fold_layer(p):
    s = p['gamma'] / jnp.sqrt(p['rv'] + 1e-5)
    t = p['beta'] - p['rm'] * s
    return (p['W'] * s[:, None]).T, jnp.tile((p['b'] * s + t)[None, :], (8, 1))


def kernel(x, batch, params):
    del batch
    blk0, blk1 = params['blocks']
    w1b0, w1c0, bias1_0 = _fold_first_layer(blk0[0], FEAT)
    w2_0, bias2_0 = _fold_layer(blk0[1])
    w3_0, bias3_0 = _fold_layer(blk0[2])
    w1b1, w1c1, bias1_1 = _fold_first_layer(blk1[0], FEAT + w1c0.shape[1])
    w2_1, bias2_1 = _fold_layer(blk1[1])
    w3_1, bias3_1 = _fold_layer(blk1[2])
    wc = params['fc1']['W'].T @ params['fc2']['W'].T
    bc = jnp.tile((params['fc1']['b'] @ params['fc2']['W'].T
                   + params['fc2']['b'])[None, :], (8, 1))

    xp = jnp.pad(x.reshape(NGRAPH, NODES, FEAT),
                 ((0, 0), (0, NPAD - NODES), (0, 0))).reshape(NGRAPH * NPAD, FEAT)

    half = NGRAPH // 2
    outs = []
    for xh in (xp[:half * NPAD], xp[half * NPAD:]):
        nbr0, b0, c0 = _stage1(xh, w1b0, w1c0, bias1_0)
        e0 = nbr0.transpose(0, 2, 1).reshape(-1)
        g0 = _gather_rows(b0, e0)
        nbr1, b1, c1, xb1 = _stage2(g0, c0, xh, w2_0, bias2_0, w3_0, bias3_0,
                                    w1b1, w1c1, bias1_1)
        e1 = nbr1.transpose(0, 2, 1).reshape(-1)
        g1 = _gather_rows(b1, e1)
        outs.append(_stage3(g1, c1, xb1, w2_1, bias2_1, w3_1, bias3_1, wc, bc))
    return jnp.concatenate(outs, axis=0)


# speedup vs baseline: 4.3557x; 1.0001x over previous
import functools

import jax
import jax.numpy as jnp
from jax import lax
from jax.experimental import pallas as pl
from jax.experimental.pallas import tpu as pltpu
from jax.experimental.pallas import tpu_sc as plsc

NGRAPH = 20
NODES = 500
NPAD = 512
KNN = 16
FEAT = 128
BIG = 1e30
_HI = lax.Precision.HIGHEST

_SC_CORES = 2
_SC_SUBCORES = 16
_SC_WORKERS = _SC_CORES * _SC_SUBCORES
_GATHER_CHUNK = 128
GROW = 128


def _dot(a, b, prec=lax.Precision.DEFAULT):
    return lax.dot_general(a, b, (((1,), (0,)), ((), ())),
                           preferred_element_type=jnp.float32, precision=prec)


def _dot_t(a, b, prec=_HI):
    return lax.dot_general(a, b, (((1,), (1,)), ((), ())),
                           preferred_element_type=jnp.float32, precision=prec)


_TKROWS = 64


def _topk_neighbors(d2_ref, nbr_ref, base):
    lane_k = lax.broadcasted_iota(jnp.int32, (_TKROWS, KNN), 1)
    int_max = jnp.int32(2 ** 31 - 1)

    for ci in range(NPAD // _TKROWS):
        key = d2_ref[ci * _TKROWS:(ci + 1) * _TKROWS, :]
        macc = jnp.zeros((_TKROWS, KNN), jnp.int32)
        for t in range(KNN):
            m = jnp.min(key, axis=1, keepdims=True)
            macc = jnp.where(lane_k == t, m, macc)
            key = jnp.where(key == m, int_max, key)
        nbr_ref[0, ci * _TKROWS:(ci + 1) * _TKROWS, :] = (macc & 511) + base


def _pair_dist(feat, d2_ref):
    hi = feat.astype(jnp.bfloat16)
    lo = (feat - hi.astype(jnp.float32)).astype(jnp.bfloat16)
    g = (_dot_t(hi, hi, prec=lax.Precision.DEFAULT)
         + (_dot_t(hi, lo, prec=lax.Precision.DEFAULT)
            + _dot_t(lo, hi, prec=lax.Precision.DEFAULT)))
    sq = jnp.sum(feat * feat, axis=1, keepdims=True)
    ones = jnp.ones((NPAD, 1), jnp.float32)
    sqr = _dot_t(ones, sq)
    d2 = jnp.maximum(sq + sqr - 2.0 * g, 0.0)
    col = lax.broadcasted_iota(jnp.int32, (NPAD, NPAD), 1)
    row = lax.broadcasted_iota(jnp.int32, (NPAD, NPAD), 0)
    d2 = jnp.where(col >= NODES, BIG, jnp.where(col == row, d2 + 1e9, d2))
    bits = lax.bitcast_convert_type(d2, jnp.int32)
    d2_ref[...] = (bits & jnp.int32(-512)) | col


def _stage1_kernel(x_ref, w1b_ref, w1c_ref, bias1_ref,
                   nbr_ref, b0_ref, c0_ref, d2_ref):
    g = pl.program_id(0)
    x = x_ref[...]
    _pair_dist(x[:, 0:2], d2_ref)
    _topk_neighbors(d2_ref, nbr_ref, g * NPAD)
    b0_ref[...] = _dot(x, w1b_ref[...]).astype(b0_ref.dtype)
    c0_ref[...] = _dot(x, w1c_ref[...]) + bias1_ref[0:1, :]


def _edge_mlp_tail(h1, w2_ref, bias2_ref, w3_ref, bias3_ref):
    h = jnp.maximum(h1, 0.0)
    h = jnp.maximum(_dot(h, w2_ref[...]) + bias2_ref[0:1, :], 0.0)
    h = jnp.maximum(_dot(h, w3_ref[...]) + bias3_ref[0:1, :], 0.0)
    acc = h[0:NPAD]
    for k in range(1, KNN):
        acc = acc + h[k * NPAD:(k + 1) * NPAD]
    return acc * (1.0 / KNN)


def _stage2_kernel(g0_ref, c0_ref, x_ref,
                   w2_ref, bias2_ref, w3_ref, bias3_ref,
                   w1b_ref, w1c_ref, bias1_ref,
                   nbr_ref, b1_ref, c1_ref, xb1_ref, d2_ref):
    g = pl.program_id(0)
    c0 = c0_ref[...]
    l0 = c0.shape[1]
    h1 = (g0_ref[:, 0:l0].astype(jnp.float32)
          + jnp.concatenate([c0] * KNN, axis=0))
    conv = _edge_mlp_tail(h1, w2_ref, bias2_ref, w3_ref, bias3_ref)
    xb1 = jnp.concatenate([conv, x_ref[...]], axis=1)
    xb1_ref[...] = xb1
    _pair_dist(xb1, d2_ref)
    _topk_neighbors(d2_ref, nbr_ref, g * NPAD)
    b1_ref[...] = _dot(xb1, w1b_ref[...]).astype(b1_ref.dtype)
    c1_ref[...] = _dot(xb1, w1c_ref[...]) + bias1_ref[0:1, :]


def _stage3_kernel(g1_ref, c1_ref, xb1_ref,
                   w2_ref, bias2_ref, w3_ref, bias3_ref,
                   wc_ref, bc_ref, out_ref):
    c1 = c1_ref[...]
    l1 = c1.shape[1]
    h1 = (g1_ref[:, 0:l1].astype(jnp.float32)
          + jnp.concatenate([c1] * KNN, axis=0))
    conv = _edge_mlp_tail(h1, w2_ref, bias2_ref, w3_ref, bias3_ref)
    xb2 = jnp.concatenate([conv, xb1_ref[...]], axis=1)
    rows = lax.broadcasted_iota(jnp.int32, (NPAD, 1), 0)
    xb2 = jnp.where(rows < NODES, xb2, 0.0)
    pooled = jnp.sum(xb2, axis=0, keepdims=True) * (1.0 / NODES)
    out_ref[0] = _dot(pooled, wc_ref[...]) + bc_ref[0:1, :]


def _graph_spec(cols):
    return pl.BlockSpec((NPAD, cols), lambda g: (g, 0))


def _edge_spec(cols):
    return pl.BlockSpec((NPAD * KNN, cols), lambda g: (g, 0))


def _full_spec(rows, cols):
    return pl.BlockSpec((rows, cols), lambda g: (0, 0))


_NBR_SPEC = pl.BlockSpec((1, NPAD, KNN), lambda g: (g, 0, 0))
_CPARAMS = pltpu.CompilerParams(dimension_semantics=("parallel",))


def _stage1(xp, w1b, w1c, bias1):
    l0 = w1c.shape[1]
    ng = xp.shape[0] // NPAD
    return pl.pallas_call(
        _stage1_kernel,
        grid=(ng,),
        in_specs=[_graph_spec(FEAT), _full_spec(FEAT, GROW),
                  _full_spec(FEAT, l0), _full_spec(8, l0)],
        out_specs=[_NBR_SPEC, _graph_spec(GROW), _graph_spec(l0)],
        out_shape=[
            jax.ShapeDtypeStruct((ng, NPAD, KNN), jnp.int32),
            jax.ShapeDtypeStruct((ng * NPAD, GROW), jnp.float32),
            jax.ShapeDtypeStruct((ng * NPAD, l0), jnp.float32),
        ],
        scratch_shapes=[pltpu.VMEM((NPAD, NPAD), jnp.int32)],
        compiler_params=_CPARAMS,
    )(xp, w1b, w1c, bias1)


def _stage2(g0, c0, xp, w2, bias2, w3, bias3, w1b, w1c, bias1):
    l0 = c0.shape[1]
    l1 = w1c.shape[1]
    d1 = FEAT + l0
    ng = c0.shape[0] // NPAD
    return pl.pallas_call(
        _stage2_kernel,
        grid=(ng,),
        in_specs=[_edge_spec(GROW), _graph_spec(l0), _graph_spec(FEAT),
                  _full_spec(l0, l0), _full_spec(8, l0),
                  _full_spec(l0, l0), _full_spec(8, l0),
                  _full_spec(d1, GROW), _full_spec(d1, l1), _full_spec(8, l1)],
        out_specs=[_NBR_SPEC, _graph_spec(GROW), _graph_spec(l1),
                   _graph_spec(d1)],
        out_shape=[
            jax.ShapeDtypeStruct((ng, NPAD, KNN), jnp.int32),
            jax.ShapeDtypeStruct((ng * NPAD, GROW), jnp.float32),
            jax.ShapeDtypeStruct((ng * NPAD, l1), jnp.float32),
            jax.ShapeDtypeStruct((ng * NPAD, d1), jnp.float32),
        ],
        scratch_shapes=[pltpu.VMEM((NPAD, NPAD), jnp.int32)],
        compiler_params=_CPARAMS,
    )(g0, c0, xp, w2, bias2, w3, bias3, w1b, w1c, bias1)


def _stage3(g1, c1, xb1, w2, bias2, w3, bias3, wc, bc):
    l1 = c1.shape[1]
    d1 = xb1.shape[1]
    d2 = d1 + l1
    ng = c1.shape[0] // NPAD
    out = pl.pallas_call(
        _stage3_kernel,
        grid=(ng,),
        in_specs=[_edge_spec(GROW), _graph_spec(l1), _graph_spec(d1),
                  _full_spec(l1, l1), _full_spec(8, l1),
                  _full_spec(l1, l1), _full_spec(8, l1),
                  _full_spec(d2, 2), _full_spec(8, 2)],
        out_specs=[pl.BlockSpec((1, 1, 2), lambda g: (g, 0, 0))],
        out_shape=[jax.ShapeDtypeStruct((ng, 1, 2), jnp.float32)],
        compiler_params=_CPARAMS,
    )(g1, c1, xb1, w2, bias2, w3, bias3, wc, bc)[0]
    return out.reshape(ng, 2)


def _make_sc_gather(v, d, b, dtype):
    per_w = b // _SC_WORKERS
    nb = per_w // _GATHER_CHUNK
    mesh = plsc.VectorSubcoreMesh(core_axis_name="c", subcore_axis_name="s")

    @functools.partial(
        pl.kernel, mesh=mesh,
        out_type=jax.ShapeDtypeStruct((b, d), dtype),
        scratch_types=[
            pltpu.VMEM((per_w,), jnp.int32),
            pltpu.VMEM((2, _GATHER_CHUNK, d), dtype),
            pltpu.SemaphoreType.DMA,
            pltpu.SemaphoreType.DMA((2,)),
            pltpu.SemaphoreType.DMA((2,)),
        ],
    )
    def gather_kernel(table_hbm, idx_hbm, out_hbm, idx_v, rows_v,
                      isem, gsem, osem):
        wid = lax.axis_index("s") * _SC_CORES + lax.axis_index("c")
        base = wid * per_w
        pltpu.make_async_copy(idx_hbm.at[pl.ds(base, per_w)], idx_v,
                              isem).start()

        def g_copy(c, buf):
            sl = idx_v.at[pl.ds(c * _GATHER_CHUNK, _GATHER_CHUNK)]
            return pltpu.make_async_copy(table_hbm.at[sl], rows_v.at[buf],
                                         gsem.at[buf])

        def o_copy(c, buf):
            dst = out_hbm.at[pl.ds(base + c * _GATHER_CHUNK, _GATHER_CHUNK)]
            return pltpu.make_async_copy(rows_v.at[buf], dst, osem.at[buf])

        pltpu.make_async_copy(idx_hbm.at[pl.ds(base, per_w)], idx_v,
                              isem).wait()
        g_copy(0, 0).start()

        @pl.loop(0, nb // 2)
        def _(i):
            for bsel in (0, 1):
                c = 2 * i + bsel

                @pl.when(c >= 1)
                def _():
                    o_copy(c - 1, 1 - bsel).wait()

                @pl.when(c + 1 < nb)
                def _():
                    g_copy(c + 1, 1 - bsel).start()

                g_copy(c, bsel).wait()
                o_copy(c, bsel).start()

        o_copy(nb - 1, (nb - 1) % 2).wait()

    return gather_kernel


def _gather_rows(table, idx):
    v, d = table.shape
    return _make_sc_gather(v, d, idx.shape[0], table.dtype)(table, idx)


def _fold_first_layer(p, feat):
    s = p['gamma'] / jnp.sqrt(p['rv'] + 1e-5)
    t = p['beta'] - p['rm'] * s
    wa = p['W'][:, :feat]
    wb = p['W'][:, feat:]
    w1b = jnp.pad((wb * s[:, None]).T, ((0, 0), (0, GROW - wb.shape[0])))
    w1c = ((wa - wb) * s[:, None]).T
    bias1 = jnp.tile((p['b'] * s + t)[None, :], (8, 1))
    return w1b, w1c, bias1


def _fold_layer(p):
    s = p['gamma'] / jnp.sqrt(p['rv'] + 1e-5)
    t = p['beta'] - p['rm'] * s
    return (p['W'] * s[:, None]).T, jnp.tile((p['b'] * s + t)[None, :], (8, 1))


def kernel(x, batch, params):
    del batch
    blk0, blk1 = params['blocks']
    w1b0, w1c0, bias1_0 = _fold_first_layer(blk0[0], FEAT)
    w2_0, bias2_0 = _fold_layer(blk0[1])
    w3_0, bias3_0 = _fold_layer(blk0[2])
    w1b1, w1c1, bias1_1 = _fold_first_layer(blk1[0], FEAT + w1c0.shape[1])
    w2_1, bias2_1 = _fold_layer(blk1[1])
    w3_1, bias3_1 = _fold_layer(blk1[2])
    wc = params['fc1']['W'].T @ params['fc2']['W'].T
    bc = jnp.tile((params['fc1']['b'] @ params['fc2']['W'].T
                   + params['fc2']['b'])[None, :], (8, 1))

    xp = jnp.pad(x.reshape(NGRAPH, NODES, FEAT),
                 ((0, 0), (0, NPAD - NODES), (0, 0))).reshape(NGRAPH * NPAD, FEAT)

    half = NGRAPH // 2
    outs = []
    for xh in (xp[:half * NPAD], xp[half * NPAD:]):
        nbr0, b0, c0 = _stage1(xh, w1b0, w1c0, bias1_0)
        e0 = nbr0.transpose(0, 2, 1).reshape(-1)
        g0 = _gather_rows(b0, e0)
        nbr1, b1, c1, xb1 = _stage2(g0, c0, xh, w2_0, bias2_0, w3_0, bias3_0,
                                    w1b1, w1c1, bias1_1)
        e1 = nbr1.transpose(0, 2, 1).reshape(-1)
        g1 = _gather_rows(b1, e1)
        outs.append(_stage3(g1, c1, xb1, w2_1, bias2_1, w3_1, bias3_1, wc, bc))
    return jnp.concatenate(outs, axis=0)
